# baseline (speedup 1.0000x reference)
import functools

import jax
import jax.numpy as jnp
from jax import lax
from jax.experimental import pallas as pl
from jax.experimental.pallas import tpu as pltpu
from jax.experimental.pallas import tpu_sc as plsc

_N = 10000
_E = 320000
_D_IN = 128
_D_HID = 512
_D_OUT = 250
_D_OUT_PAD = 256

_NC = 2
_NS = 16
_CHUNK = 64
_IBLK = 16
_NBUF = 2
_EP = 327680
_NTRASH = 512
_NA = _N + _NTRASH

_DRAIN_TILES = 10
_DRAIN_ROWS = _N // _DRAIN_TILES

_MBLK = 1000

_mesh = plsc.VectorSubcoreMesh(core_axis_name="c", subcore_axis_name="s")


def _init_acc(zeros_hbm, acc, s):
    @pl.when(s < _DRAIN_TILES)
    def _():
        rbase = s * _DRAIN_ROWS
        pltpu.sync_copy(zeros_hbm.at[pl.ds(rbase, _DRAIN_ROWS)],
                        acc.at[pl.ds(rbase, _DRAIN_ROWS)])

    @pl.when(s == _DRAIN_TILES)
    def _():
        pltpu.sync_copy(zeros_hbm.at[pl.ds(_N, _NA - _N)],
                        acc.at[pl.ds(_N, _NA - _N)])


def _drain_acc(acc, out0, out1, c, s):
    @pl.when(jnp.logical_and(c == 0, s < _DRAIN_TILES))
    def _():
        rbase = s * _DRAIN_ROWS
        pltpu.sync_copy(acc.at[pl.ds(rbase, _DRAIN_ROWS)],
                        out0.at[pl.ds(rbase, _DRAIN_ROWS)])

    @pl.when(jnp.logical_and(c == 1, s < _DRAIN_TILES))
    def _():
        rbase = s * _DRAIN_ROWS
        pltpu.sync_copy(acc.at[pl.ds(rbase, _DRAIN_ROWS)],
                        out1.at[pl.ds(rbase, _DRAIN_ROWS)])


def _make_agg(data_rows, n_chunks, dst_split_cores):
    @functools.partial(
        pl.kernel,
        mesh=_mesh,
        out_type=(
            jax.ShapeDtypeStruct((_N, 128), jnp.float32),
            jax.ShapeDtypeStruct((_N, 128), jnp.float32),
        ),
        scratch_types=(
            [pltpu.VMEM((_IBLK, _CHUNK), jnp.int32),
             pltpu.VMEM((_IBLK, _CHUNK), jnp.int32)]
            + [pltpu.VMEM((_CHUNK, 128), jnp.float32)] * _NBUF
            + [pltpu.SemaphoreType.DMA] * (2 * _NBUF)
            + [pltpu.VMEM_SHARED((_NA, 128), jnp.float32)]
        ),
    )
    def agg(data_hbm, src_hbm, dst_hbm, zeros_hbm, out0, out1,
            sidx, didx, *rest):
        bufs = rest[:_NBUF]
        gsems = rest[_NBUF:2 * _NBUF]
        ssems = rest[2 * _NBUF:3 * _NBUF]
        acc = rest[3 * _NBUF]
        c = lax.axis_index("c")
        s = lax.axis_index("s")
        _init_acc(zeros_hbm, acc, s)
        sbase = (c * _NS + s) * n_chunks
        dbase = ((c * _NS * n_chunks) if dst_split_cores else 0) + s * n_chunks
        plsc.subcore_barrier()

        def start_g(j, i):
            pltpu.async_copy(data_hbm.at[sidx.at[j]], bufs[i], gsems[i])

        def wait_g(j, i):
            pltpu.make_async_copy(data_hbm.at[sidx.at[j]], bufs[i],
                                  gsems[i]).wait()

        def start_s(j, i):
            pltpu.async_copy(bufs[i], acc.at[didx.at[j]], ssems[i], add=True)

        def wait_s(j, i):
            pltpu.make_async_copy(bufs[i], acc.at[didx.at[j]],
                                  ssems[i]).wait()

        @pl.loop(0, n_chunks // _IBLK)
        def _(b):
            pltpu.sync_copy(src_hbm.at[pl.ds(sbase + b * _IBLK, _IBLK)], sidx)
            pltpu.sync_copy(dst_hbm.at[pl.ds(dbase + b * _IBLK, _IBLK)], didx)
            for i in range(_NBUF):
                start_g(i, i)

            @pl.loop(0, _IBLK // _NBUF - 1)
            def _(k):
                j = _NBUF * k
                for i in range(_NBUF):
                    wait_g(j + i, i)
                    start_s(j + i, i)
                for i in range(_NBUF):
                    wait_s(j + i, i)
                    start_g(j + _NBUF + i, i)

            jt = _IBLK - _NBUF
            for i in range(_NBUF):
                wait_g(jt + i, i)
                start_s(jt + i, i)
            for i in range(_NBUF):
                wait_s(jt + i, i)

        plsc.subcore_barrier()
        _drain_acc(acc, out0, out1, c, s)

    return agg


_agg1 = _make_agg(_N, (_EP // 2) // _NS // _CHUNK, True)
_agg2 = _make_agg(2 * _N, _EP // _NS // _CHUNK, False)

_DEG_CHUNKS = (_EP // 2) // _NS // _CHUNK


@functools.partial(
    pl.kernel,
    mesh=_mesh,
    out_type=(
        jax.ShapeDtypeStruct((_N, 128), jnp.float32),
        jax.ShapeDtypeStruct((_N, 128), jnp.float32),
    ),
    scratch_types=[
        pltpu.VMEM((_IBLK, _CHUNK), jnp.int32),
        pltpu.VMEM((_CHUNK, 128), jnp.float32),
        pltpu.SemaphoreType.DMA,
        pltpu.VMEM_SHARED((_NA, 128), jnp.float32),
    ],
)
def _deg(dst_hbm, zeros_hbm, ones_hbm, out0, out1, didx, ones_v, sem, acc):
    c = lax.axis_index("c")
    s = lax.axis_index("s")
    _init_acc(zeros_hbm, acc, s)
    pltpu.sync_copy(ones_hbm, ones_v)
    dbase = (c * _NS + s) * _DEG_CHUNKS
    plsc.subcore_barrier()

    @pl.loop(0, _DEG_CHUNKS // _IBLK)
    def _(b):
        pltpu.sync_copy(dst_hbm.at[pl.ds(dbase + b * _IBLK, _IBLK)], didx)

        @pl.loop(0, _IBLK)
        def _(j):
            pltpu.async_copy(ones_v, acc.at[didx.at[j]], sem, add=True)

        @pl.loop(0, _IBLK)
        def _(j):
            pltpu.make_async_copy(ones_v, acc.at[didx.at[j]], sem).wait()

    plsc.subcore_barrier()
    _drain_acc(acc, out0, out1, c, s)


def _scale_body(d0, d1, x, p):
    deg = d0[:, :1] + d1[:, :1] + 1.0
    p[...] = x[...] * lax.rsqrt(deg)


def _mid_body(a0, a1, p, d0, d1, w1, b1, w2, q):
    r = lax.rsqrt(d0[:, :1] + d1[:, :1] + 1.0)
    u = (a0[...] + a1[...] + p[...]) * r
    h = jnp.dot(u, w1[...], preferred_element_type=jnp.float32) + b1[...]
    h = jnp.maximum(h, 0.0)
    q[...] = jnp.dot(h, w2[...], preferred_element_type=jnp.float32) * r


def _final_body(a0, a1, q, d0, d1, b2, o):
    r = lax.rsqrt(d0[:, :1] + d1[:, :1] + 1.0)
    acc = jnp.concatenate([a0[...], a1[...]], axis=1) + q[...]
    full = jnp.maximum(acc * r + b2[...], 0.0)
    o[...] = full[:, :_D_OUT]


def _row_spec(w):
    return pl.BlockSpec((_MBLK, w), lambda i: (i, 0))


def _full_spec(h, w):
    return pl.BlockSpec((h, w), lambda i: (0, 0))


_GRID = (_N // _MBLK,)

_scale = pl.pallas_call(
    _scale_body,
    grid=_GRID,
    in_specs=[_row_spec(128), _row_spec(128), _row_spec(128)],
    out_specs=_row_spec(128),
    out_shape=jax.ShapeDtypeStruct((_N, 128), jnp.float32),
)

_mid = pl.pallas_call(
    _mid_body,
    grid=_GRID,
    in_specs=[
        _row_spec(128), _row_spec(128), _row_spec(128),
        _row_spec(128), _row_spec(128),
        _full_spec(_D_IN, _D_HID), _full_spec(1, _D_HID),
        _full_spec(_D_HID, _D_OUT_PAD),
    ],
    out_specs=_row_spec(_D_OUT_PAD),
    out_shape=jax.ShapeDtypeStruct((_N, _D_OUT_PAD), jnp.float32),
)

_final = pl.pallas_call(
    _final_body,
    grid=_GRID,
    in_specs=[
        _row_spec(128), _row_spec(128), _row_spec(_D_OUT_PAD),
        _row_spec(128), _row_spec(128),
        _full_spec(1, _D_OUT_PAD),
    ],
    out_specs=_row_spec(_D_OUT),
    out_shape=jax.ShapeDtypeStruct((_N, _D_OUT), jnp.float32),
)


def kernel(x, edge_index, W1, b1, W2, b2):
    src = edge_index[0].astype(jnp.int32)
    dst = edge_index[1].astype(jnp.int32)
    pad = _EP - _E
    spread = jnp.arange(pad, dtype=jnp.int32)
    src_p = jnp.concatenate([src, spread % _N])
    dst_p = jnp.concatenate([dst, _N + spread % _NTRASH])
    src_r = src_p.reshape(_EP // _CHUNK, _CHUNK)
    dst_r = dst_p.reshape(_EP // _CHUNK, _CHUNK)
    src2_r = jnp.concatenate([2 * src_p, 2 * src_p + 1]).reshape(
        2 * _EP // _CHUNK, _CHUNK)
    zeros128 = jnp.zeros((_NA, 128), jnp.float32)
    ones128 = jnp.ones((_CHUNK, 128), jnp.float32)

    d0, d1 = _deg(dst_r, zeros128, ones128)
    p = _scale(d0, d1, x)
    a0, a1 = _agg1(p, src_r, dst_r, zeros128)
    q2 = _mid(a0, a1, p, d0, d1, W1, b1.reshape(1, _D_HID),
              jnp.pad(W2, ((0, 0), (0, _D_OUT_PAD - _D_OUT))))
    q_all = q2.reshape(2 * _N, 128)
    c0, c1 = _agg2(q_all, src2_r, dst_r, zeros128)
    out = _final(c0, c1, q2, d0, d1,
                 jnp.pad(b2, (0, _D_OUT_PAD - _D_OUT)).reshape(1, _D_OUT_PAD))
    return out


# speedup vs baseline: 1.2073x; 1.2073x over previous
import functools

import jax
import jax.numpy as jnp
from jax import lax
from jax.experimental import pallas as pl
from jax.experimental.pallas import tpu as pltpu
from jax.experimental.pallas import tpu_sc as plsc

_N = 10000
_E = 320000
_D_IN = 128
_D_HID = 512
_D_OUT = 250
_D_OUT_PAD = 256

_NC = 2
_NS = 16
_CHUNK = 32
_IBLK = 64
_NBUF = 4
_EP = 327680
_NTRASH = 512
_NA = _N + _NTRASH

_DRAIN_TILES = 10
_DRAIN_ROWS = _N // _DRAIN_TILES

_MBLK = 1000

_mesh = plsc.VectorSubcoreMesh(core_axis_name="c", subcore_axis_name="s")


def _init_acc(zeros_hbm, acc, s):
    @pl.when(s < _DRAIN_TILES)
    def _():
        rbase = s * _DRAIN_ROWS
        pltpu.sync_copy(zeros_hbm.at[pl.ds(rbase, _DRAIN_ROWS)],
                        acc.at[pl.ds(rbase, _DRAIN_ROWS)])

    @pl.when(s == _DRAIN_TILES)
    def _():
        pltpu.sync_copy(zeros_hbm.at[pl.ds(_N, _NA - _N)],
                        acc.at[pl.ds(_N, _NA - _N)])


def _drain_acc(acc, out0, out1, c, s):
    @pl.when(jnp.logical_and(c == 0, s < _DRAIN_TILES))
    def _():
        rbase = s * _DRAIN_ROWS
        pltpu.sync_copy(acc.at[pl.ds(rbase, _DRAIN_ROWS)],
                        out0.at[pl.ds(rbase, _DRAIN_ROWS)])

    @pl.when(jnp.logical_and(c == 1, s < _DRAIN_TILES))
    def _():
        rbase = s * _DRAIN_ROWS
        pltpu.sync_copy(acc.at[pl.ds(rbase, _DRAIN_ROWS)],
                        out1.at[pl.ds(rbase, _DRAIN_ROWS)])


def _make_agg(data_rows, n_chunks, dst_split_cores):
    @functools.partial(
        pl.kernel,
        mesh=_mesh,
        out_type=(
            jax.ShapeDtypeStruct((_N, 128), jnp.float32),
            jax.ShapeDtypeStruct((_N, 128), jnp.float32),
        ),
        scratch_types=(
            [pltpu.VMEM((_IBLK, _CHUNK), jnp.int32),
             pltpu.VMEM((_IBLK, _CHUNK), jnp.int32)]
            + [pltpu.VMEM((_CHUNK, 128), jnp.float32)] * _NBUF
            + [pltpu.SemaphoreType.DMA] * (2 * _NBUF)
            + [pltpu.VMEM_SHARED((_NA, 128), jnp.float32)]
        ),
    )
    def agg(data_hbm, src_hbm, dst_hbm, zeros_hbm, out0, out1,
            sidx, didx, *rest):
        bufs = rest[:_NBUF]
        gsems = rest[_NBUF:2 * _NBUF]
        ssems = rest[2 * _NBUF:3 * _NBUF]
        acc = rest[3 * _NBUF]
        c = lax.axis_index("c")
        s = lax.axis_index("s")
        _init_acc(zeros_hbm, acc, s)
        sbase = (c * _NS + s) * n_chunks
        dbase = ((c * _NS * n_chunks) if dst_split_cores else 0) + s * n_chunks
        plsc.subcore_barrier()

        def start_g(j, i):
            pltpu.async_copy(data_hbm.at[sidx.at[j]], bufs[i], gsems[i])

        def wait_g(j, i):
            pltpu.make_async_copy(data_hbm.at[sidx.at[j]], bufs[i],
                                  gsems[i]).wait()

        def start_s(j, i):
            pltpu.async_copy(bufs[i], acc.at[didx.at[j]], ssems[i], add=True)

        def wait_s(j, i):
            pltpu.make_async_copy(bufs[i], acc.at[didx.at[j]],
                                  ssems[i]).wait()

        @pl.loop(0, n_chunks // _IBLK)
        def _(b):
            pltpu.sync_copy(src_hbm.at[pl.ds(sbase + b * _IBLK, _IBLK)], sidx)
            pltpu.sync_copy(dst_hbm.at[pl.ds(dbase + b * _IBLK, _IBLK)], didx)
            for i in range(_NBUF):
                start_g(i, i)

            @pl.loop(0, _IBLK // _NBUF - 1)
            def _(k):
                j = _NBUF * k
                for i in range(_NBUF):
                    wait_g(j + i, i)
                    start_s(j + i, i)
                for i in range(_NBUF):
                    wait_s(j + i, i)
                    start_g(j + _NBUF + i, i)

            jt = _IBLK - _NBUF
            for i in range(_NBUF):
                wait_g(jt + i, i)
                start_s(jt + i, i)
            for i in range(_NBUF):
                wait_s(jt + i, i)

        plsc.subcore_barrier()
        _drain_acc(acc, out0, out1, c, s)

    return agg


_agg1 = _make_agg(_N, (_EP // 2) // _NS // _CHUNK, True)
_agg2 = _make_agg(2 * _N, _EP // _NS // _CHUNK, False)

_DEG_CHUNKS = (_EP // 2) // _NS // _CHUNK


@functools.partial(
    pl.kernel,
    mesh=_mesh,
    out_type=(
        jax.ShapeDtypeStruct((_N, 128), jnp.float32),
        jax.ShapeDtypeStruct((_N, 128), jnp.float32),
    ),
    scratch_types=[
        pltpu.VMEM((_IBLK, _CHUNK), jnp.int32),
        pltpu.VMEM((_CHUNK, 128), jnp.float32),
        pltpu.SemaphoreType.DMA,
        pltpu.VMEM_SHARED((_NA, 128), jnp.float32),
    ],
)
def _deg(dst_hbm, zeros_hbm, ones_hbm, out0, out1, didx, ones_v, sem, acc):
    c = lax.axis_index("c")
    s = lax.axis_index("s")
    _init_acc(zeros_hbm, acc, s)
    pltpu.sync_copy(ones_hbm, ones_v)
    dbase = (c * _NS + s) * _DEG_CHUNKS
    plsc.subcore_barrier()

    @pl.loop(0, _DEG_CHUNKS // _IBLK)
    def _(b):
        pltpu.sync_copy(dst_hbm.at[pl.ds(dbase + b * _IBLK, _IBLK)], didx)

        @pl.loop(0, _IBLK)
        def _(j):
            pltpu.async_copy(ones_v, acc.at[didx.at[j]], sem, add=True)

        @pl.loop(0, _IBLK)
        def _(j):
            pltpu.make_async_copy(ones_v, acc.at[didx.at[j]], sem).wait()

    plsc.subcore_barrier()
    _drain_acc(acc, out0, out1, c, s)


def _scale_body(d0, d1, x, p):
    deg = d0[:, :1] + d1[:, :1] + 1.0
    p[...] = x[...] * lax.rsqrt(deg)


def _mid_body(a0, a1, p, d0, d1, w1, b1, w2, q):
    r = lax.rsqrt(d0[:, :1] + d1[:, :1] + 1.0)
    u = (a0[...] + a1[...] + p[...]) * r
    h = jnp.dot(u, w1[...], preferred_element_type=jnp.float32) + b1[...]
    h = jnp.maximum(h, 0.0)
    q[...] = jnp.dot(h, w2[...], preferred_element_type=jnp.float32) * r


def _final_body(a0, a1, q, d0, d1, b2, o):
    r = lax.rsqrt(d0[:, :1] + d1[:, :1] + 1.0)
    acc = jnp.concatenate([a0[...], a1[...]], axis=1) + q[...]
    full = jnp.maximum(acc * r + b2[...], 0.0)
    o[...] = full[:, :_D_OUT]


def _row_spec(w):
    return pl.BlockSpec((_MBLK, w), lambda i: (i, 0))


def _full_spec(h, w):
    return pl.BlockSpec((h, w), lambda i: (0, 0))


_GRID = (_N // _MBLK,)

_scale = pl.pallas_call(
    _scale_body,
    grid=_GRID,
    in_specs=[_row_spec(128), _row_spec(128), _row_spec(128)],
    out_specs=_row_spec(128),
    out_shape=jax.ShapeDtypeStruct((_N, 128), jnp.float32),
)

_mid = pl.pallas_call(
    _mid_body,
    grid=_GRID,
    in_specs=[
        _row_spec(128), _row_spec(128), _row_spec(128),
        _row_spec(128), _row_spec(128),
        _full_spec(_D_IN, _D_HID), _full_spec(1, _D_HID),
        _full_spec(_D_HID, _D_OUT_PAD),
    ],
    out_specs=_row_spec(_D_OUT_PAD),
    out_shape=jax.ShapeDtypeStruct((_N, _D_OUT_PAD), jnp.float32),
)

_final = pl.pallas_call(
    _final_body,
    grid=_GRID,
    in_specs=[
        _row_spec(128), _row_spec(128), _row_spec(_D_OUT_PAD),
        _row_spec(128), _row_spec(128),
        _full_spec(1, _D_OUT_PAD),
    ],
    out_specs=_row_spec(_D_OUT),
    out_shape=jax.ShapeDtypeStruct((_N, _D_OUT), jnp.float32),
)


def kernel(x, edge_index, W1, b1, W2, b2):
    src = edge_index[0].astype(jnp.int32)
    dst = edge_index[1].astype(jnp.int32)
    pad = _EP - _E
    spread = jnp.arange(pad, dtype=jnp.int32)
    src_p = jnp.concatenate([src, spread % _N])
    dst_p = jnp.concatenate([dst, _N + spread % _NTRASH])
    src_r = src_p.reshape(_EP // _CHUNK, _CHUNK)
    dst_r = dst_p.reshape(_EP // _CHUNK, _CHUNK)
    src2_r = jnp.concatenate([2 * src_p, 2 * src_p + 1]).reshape(
        2 * _EP // _CHUNK, _CHUNK)
    zeros128 = jnp.zeros((_NA, 128), jnp.float32)
    ones128 = jnp.ones((_CHUNK, 128), jnp.float32)

    d0, d1 = _deg(dst_r, zeros128, ones128)
    p = _scale(d0, d1, x)
    a0, a1 = _agg1(p, src_r, dst_r, zeros128)
    q2 = _mid(a0, a1, p, d0, d1, W1, b1.reshape(1, _D_HID),
              jnp.pad(W2, ((0, 0), (0, _D_OUT_PAD - _D_OUT))))
    q_all = q2.reshape(2 * _N, 128)
    c0, c1 = _agg2(q_all, src2_r, dst_r, zeros128)
    out = _final(c0, c1, q2, d0, d1,
                 jnp.pad(b2, (0, _D_OUT_PAD - _D_OUT)).reshape(1, _D_OUT_PAD))
    return out


# speedup vs baseline: 1.2328x; 1.0211x over previous
import functools

import jax
import jax.numpy as jnp
from jax import lax
from jax.experimental import pallas as pl
from jax.experimental.pallas import tpu as pltpu
from jax.experimental.pallas import tpu_sc as plsc

_N = 10000
_E = 320000
_D_IN = 128
_D_HID = 512
_D_OUT = 250
_D_OUT_PAD = 256

_NC = 2
_NS = 16
_CHUNK = 32
_IBLK = 64
_NBUF = 4
_EP = 327680
_NTRASH = 512
_NA = _N + _NTRASH

_DRAIN_TILES = 10
_DRAIN_ROWS = _N // _DRAIN_TILES

_MBLK = 1000

_mesh = plsc.VectorSubcoreMesh(core_axis_name="c", subcore_axis_name="s")


def _init_acc(zeros_hbm, acc, s):
    @pl.when(s < _DRAIN_TILES)
    def _():
        rbase = s * _DRAIN_ROWS
        pltpu.sync_copy(zeros_hbm.at[pl.ds(rbase, _DRAIN_ROWS)],
                        acc.at[pl.ds(rbase, _DRAIN_ROWS)])

    @pl.when(s == _DRAIN_TILES)
    def _():
        pltpu.sync_copy(zeros_hbm.at[pl.ds(_N, _NA - _N)],
                        acc.at[pl.ds(_N, _NA - _N)])


def _drain_acc(acc, out0, out1, c, s):
    @pl.when(jnp.logical_and(c == 0, s < _DRAIN_TILES))
    def _():
        rbase = s * _DRAIN_ROWS
        pltpu.sync_copy(acc.at[pl.ds(rbase, _DRAIN_ROWS)],
                        out0.at[pl.ds(rbase, _DRAIN_ROWS)])

    @pl.when(jnp.logical_and(c == 1, s < _DRAIN_TILES))
    def _():
        rbase = s * _DRAIN_ROWS
        pltpu.sync_copy(acc.at[pl.ds(rbase, _DRAIN_ROWS)],
                        out1.at[pl.ds(rbase, _DRAIN_ROWS)])


def _make_agg(data_rows, n_chunks, dst_split_cores):
    @functools.partial(
        pl.kernel,
        mesh=_mesh,
        out_type=(
            jax.ShapeDtypeStruct((_N, 128), jnp.float32),
            jax.ShapeDtypeStruct((_N, 128), jnp.float32),
        ),
        scratch_types=(
            [pltpu.VMEM((_IBLK, _CHUNK), jnp.int32),
             pltpu.VMEM((_IBLK, _CHUNK), jnp.int32)]
            + [pltpu.VMEM((_CHUNK, 128), jnp.float32)] * _NBUF
            + [pltpu.SemaphoreType.DMA] * (2 * _NBUF)
            + [pltpu.VMEM_SHARED((_NA, 128), jnp.float32)]
        ),
    )
    def agg(data_hbm, src_hbm, dst_hbm, zeros_hbm, out0, out1,
            sidx, didx, *rest):
        bufs = rest[:_NBUF]
        gsems = rest[_NBUF:2 * _NBUF]
        ssems = rest[2 * _NBUF:3 * _NBUF]
        acc = rest[3 * _NBUF]
        c = lax.axis_index("c")
        s = lax.axis_index("s")
        _init_acc(zeros_hbm, acc, s)
        sbase = (c * _NS + s) * n_chunks
        dbase = ((c * _NS * n_chunks) if dst_split_cores else 0) + s * n_chunks
        plsc.subcore_barrier()

        def start_g(j, i):
            pltpu.async_copy(data_hbm.at[sidx.at[j]], bufs[i], gsems[i])

        def wait_g(j, i):
            pltpu.make_async_copy(data_hbm.at[sidx.at[j]], bufs[i],
                                  gsems[i]).wait()

        def start_s(j, i):
            pltpu.async_copy(bufs[i], acc.at[didx.at[j]], ssems[i], add=True)

        def wait_s(j, i):
            pltpu.make_async_copy(bufs[i], acc.at[didx.at[j]],
                                  ssems[i]).wait()

        @pl.loop(0, n_chunks // _IBLK)
        def _(b):
            pltpu.sync_copy(src_hbm.at[pl.ds(sbase + b * _IBLK, _IBLK)], sidx)
            pltpu.sync_copy(dst_hbm.at[pl.ds(dbase + b * _IBLK, _IBLK)], didx)
            for i in range(_NBUF):
                start_g(i, i)

            @pl.loop(0, _IBLK // _NBUF - 1)
            def _(k):
                j = _NBUF * k
                for i in range(_NBUF):
                    wait_g(j + i, i)
                    start_s(j + i, i)
                for i in range(_NBUF):
                    wait_s(j + i, i)
                    start_g(j + _NBUF + i, i)

            jt = _IBLK - _NBUF
            for i in range(_NBUF):
                wait_g(jt + i, i)
                start_s(jt + i, i)
            for i in range(_NBUF):
                wait_s(jt + i, i)

        plsc.subcore_barrier()
        _drain_acc(acc, out0, out1, c, s)

    return agg


_agg1 = _make_agg(_N, (_EP // 2) // _NS // _CHUNK, True)
_agg2 = _make_agg(2 * _N, _EP // _NS // _CHUNK, False)

_DEG_CHUNKS = (_EP // 2) // _NS // _CHUNK


@functools.partial(
    pl.kernel,
    mesh=_mesh,
    out_type=(
        jax.ShapeDtypeStruct((_N, 128), jnp.float32),
        jax.ShapeDtypeStruct((_N, 128), jnp.float32),
    ),
    scratch_types=[
        pltpu.VMEM((_IBLK, _CHUNK), jnp.int32),
        pltpu.VMEM((_CHUNK, 128), jnp.float32),
        pltpu.SemaphoreType.DMA,
        pltpu.VMEM_SHARED((_NA, 128), jnp.float32),
    ],
)
def _deg(dst_hbm, zeros_hbm, ones_hbm, out0, out1, didx, ones_v, sem, acc):
    c = lax.axis_index("c")
    s = lax.axis_index("s")
    _init_acc(zeros_hbm, acc, s)
    pltpu.sync_copy(ones_hbm, ones_v)
    dbase = (c * _NS + s) * _DEG_CHUNKS
    plsc.subcore_barrier()

    @pl.loop(0, _DEG_CHUNKS // _IBLK)
    def _(b):
        pltpu.sync_copy(dst_hbm.at[pl.ds(dbase + b * _IBLK, _IBLK)], didx)

        @pl.loop(0, _IBLK)
        def _(j):
            pltpu.async_copy(ones_v, acc.at[didx.at[j]], sem, add=True)

        @pl.loop(0, _IBLK)
        def _(j):
            pltpu.make_async_copy(ones_v, acc.at[didx.at[j]], sem).wait()

    plsc.subcore_barrier()
    _drain_acc(acc, out0, out1, c, s)


def _scale_body(d0, d1, x, p):
    deg = d0[:, :1] + d1[:, :1] + 1.0
    p[...] = x[...] * lax.rsqrt(deg)


def _mid_body(a0, a1, p, d0, d1, w1, b1, w2, q):
    r = lax.rsqrt(d0[:, :1] + d1[:, :1] + 1.0)
    u = (a0[...] + a1[...] + p[...]) * r
    h = jnp.dot(u, w1[...], preferred_element_type=jnp.float32) + b1[...]
    h = jnp.maximum(h, 0.0)
    qv = jnp.dot(h, w2[...], preferred_element_type=jnp.float32) * r
    q[...] = qv.reshape(2 * _MBLK, 128)


def _final_body(a0, a1, q, d0, d1, b2, o):
    r = lax.rsqrt(d0[:, :1] + d1[:, :1] + 1.0)
    qv = q[...].reshape(_MBLK, _D_OUT_PAD)
    acc = jnp.concatenate([a0[...], a1[...]], axis=1) + qv
    full = jnp.maximum(acc * r + b2[...], 0.0)
    o[...] = full[:, :_D_OUT]


def _row_spec(w):
    return pl.BlockSpec((_MBLK, w), lambda i: (i, 0))


def _full_spec(h, w):
    return pl.BlockSpec((h, w), lambda i: (0, 0))


_GRID = (_N // _MBLK,)

_scale = pl.pallas_call(
    _scale_body,
    grid=_GRID,
    in_specs=[_row_spec(128), _row_spec(128), _row_spec(128)],
    out_specs=_row_spec(128),
    out_shape=jax.ShapeDtypeStruct((_N, 128), jnp.float32),
)

_mid = pl.pallas_call(
    _mid_body,
    grid=_GRID,
    in_specs=[
        _row_spec(128), _row_spec(128), _row_spec(128),
        _row_spec(128), _row_spec(128),
        _full_spec(_D_IN, _D_HID), _full_spec(1, _D_HID),
        _full_spec(_D_HID, _D_OUT_PAD),
    ],
    out_specs=pl.BlockSpec((2 * _MBLK, 128), lambda i: (i, 0)),
    out_shape=jax.ShapeDtypeStruct((2 * _N, 128), jnp.float32),
)

_final = pl.pallas_call(
    _final_body,
    grid=_GRID,
    in_specs=[
        _row_spec(128), _row_spec(128),
        pl.BlockSpec((2 * _MBLK, 128), lambda i: (i, 0)),
        _row_spec(128), _row_spec(128),
        _full_spec(1, _D_OUT_PAD),
    ],
    out_specs=_row_spec(_D_OUT),
    out_shape=jax.ShapeDtypeStruct((_N, _D_OUT), jnp.float32),
)


def kernel(x, edge_index, W1, b1, W2, b2):
    src = edge_index[0].astype(jnp.int32)
    dst = edge_index[1].astype(jnp.int32)
    pad = _EP - _E
    spread = jnp.arange(pad, dtype=jnp.int32)
    src_p = jnp.concatenate([src, spread % _N])
    dst_p = jnp.concatenate([dst, _N + spread % _NTRASH])
    src_r = src_p.reshape(_EP // _CHUNK, _CHUNK)
    dst_r = dst_p.reshape(_EP // _CHUNK, _CHUNK)
    src2_r = jnp.concatenate([2 * src_p, 2 * src_p + 1]).reshape(
        2 * _EP // _CHUNK, _CHUNK)
    zeros128 = jnp.zeros((_NA, 128), jnp.float32)
    ones128 = jnp.ones((_CHUNK, 128), jnp.float32)

    d0, d1 = _deg(dst_r, zeros128, ones128)
    p = _scale(d0, d1, x)
    a0, a1 = _agg1(p, src_r, dst_r, zeros128)
    q_all = _mid(a0, a1, p, d0, d1, W1, b1.reshape(1, _D_HID),
                 jnp.pad(W2, ((0, 0), (0, _D_OUT_PAD - _D_OUT))))
    c0, c1 = _agg2(q_all, src2_r, dst_r, zeros128)
    out = _final(c0, c1, q_all, d0, d1,
                 jnp.pad(b2, (0, _D_OUT_PAD - _D_OUT)).reshape(1, _D_OUT_PAD))
    return out


# speedup vs baseline: 1.2467x; 1.0113x over previous
import functools

import jax
import jax.numpy as jnp
from jax import lax
from jax.experimental import pallas as pl
from jax.experimental.pallas import tpu as pltpu
from jax.experimental.pallas import tpu_sc as plsc

_N = 10000
_E = 320000
_D_IN = 128
_D_HID = 512
_D_OUT = 250
_D_OUT_PAD = 256

_NC = 2
_NS = 16
_CHUNK = 32
_IBLK = 40
_NBUF = 5
_EP = 327680
_NTRASH = 128
_NA = _N + _NTRASH

_DRAIN_TILES = 10
_DRAIN_ROWS = _N // _DRAIN_TILES

_MBLK = 1000

_mesh = plsc.VectorSubcoreMesh(core_axis_name="c", subcore_axis_name="s")


def _init_acc(zeros_hbm, acc, s):
    @pl.when(s < _DRAIN_TILES)
    def _():
        rbase = s * _DRAIN_ROWS
        pltpu.sync_copy(zeros_hbm.at[pl.ds(rbase, _DRAIN_ROWS)],
                        acc.at[pl.ds(rbase, _DRAIN_ROWS)])

    @pl.when(s == _DRAIN_TILES)
    def _():
        pltpu.sync_copy(zeros_hbm.at[pl.ds(_N, _NA - _N)],
                        acc.at[pl.ds(_N, _NA - _N)])


def _drain_acc(acc, out0, out1, c, s):
    @pl.when(jnp.logical_and(c == 0, s < _DRAIN_TILES))
    def _():
        rbase = s * _DRAIN_ROWS
        pltpu.sync_copy(acc.at[pl.ds(rbase, _DRAIN_ROWS)],
                        out0.at[pl.ds(rbase, _DRAIN_ROWS)])

    @pl.when(jnp.logical_and(c == 1, s < _DRAIN_TILES))
    def _():
        rbase = s * _DRAIN_ROWS
        pltpu.sync_copy(acc.at[pl.ds(rbase, _DRAIN_ROWS)],
                        out1.at[pl.ds(rbase, _DRAIN_ROWS)])


def _make_agg(data_rows, n_chunks, dst_split_cores):
    @functools.partial(
        pl.kernel,
        mesh=_mesh,
        out_type=(
            jax.ShapeDtypeStruct((_N, 128), jnp.float32),
            jax.ShapeDtypeStruct((_N, 128), jnp.float32),
        ),
        scratch_types=(
            [pltpu.VMEM((_IBLK, _CHUNK), jnp.int32),
             pltpu.VMEM((_IBLK, _CHUNK), jnp.int32)]
            + [pltpu.VMEM((_CHUNK, 128), jnp.float32)] * _NBUF
            + [pltpu.SemaphoreType.DMA] * (2 * _NBUF)
            + [pltpu.VMEM_SHARED((_NA, 128), jnp.float32)]
        ),
    )
    def agg(data_hbm, src_hbm, dst_hbm, zeros_hbm, out0, out1,
            sidx, didx, *rest):
        bufs = rest[:_NBUF]
        gsems = rest[_NBUF:2 * _NBUF]
        ssems = rest[2 * _NBUF:3 * _NBUF]
        acc = rest[3 * _NBUF]
        c = lax.axis_index("c")
        s = lax.axis_index("s")
        _init_acc(zeros_hbm, acc, s)
        sbase = (c * _NS + s) * n_chunks
        dbase = ((c * _NS * n_chunks) if dst_split_cores else 0) + s * n_chunks
        plsc.subcore_barrier()

        def start_g(j, i):
            pltpu.async_copy(data_hbm.at[sidx.at[j]], bufs[i], gsems[i])

        def wait_g(j, i):
            pltpu.make_async_copy(data_hbm.at[sidx.at[j]], bufs[i],
                                  gsems[i]).wait()

        def start_s(j, i):
            pltpu.async_copy(bufs[i], acc.at[didx.at[j]], ssems[i], add=True)

        def wait_s(j, i):
            pltpu.make_async_copy(bufs[i], acc.at[didx.at[j]],
                                  ssems[i]).wait()

        @pl.loop(0, n_chunks // _IBLK)
        def _(b):
            pltpu.sync_copy(src_hbm.at[pl.ds(sbase + b * _IBLK, _IBLK)], sidx)
            pltpu.sync_copy(dst_hbm.at[pl.ds(dbase + b * _IBLK, _IBLK)], didx)
            for i in range(_NBUF):
                start_g(i, i)

            @pl.loop(0, _IBLK // _NBUF - 1)
            def _(k):
                j = _NBUF * k
                for i in range(_NBUF):
                    wait_g(j + i, i)
                    start_s(j + i, i)
                for i in range(_NBUF):
                    wait_s(j + i, i)
                    start_g(j + _NBUF + i, i)

            jt = _IBLK - _NBUF
            for i in range(_NBUF):
                wait_g(jt + i, i)
                start_s(jt + i, i)
            for i in range(_NBUF):
                wait_s(jt + i, i)

        plsc.subcore_barrier()
        _drain_acc(acc, out0, out1, c, s)

    return agg


_agg1 = _make_agg(_N, (_EP // 2) // _NS // _CHUNK, True)
_agg2 = _make_agg(2 * _N, _EP // _NS // _CHUNK, False)

_DEG_CHUNKS = (_EP // 2) // _NS // _CHUNK


@functools.partial(
    pl.kernel,
    mesh=_mesh,
    out_type=(
        jax.ShapeDtypeStruct((_N, 128), jnp.float32),
        jax.ShapeDtypeStruct((_N, 128), jnp.float32),
    ),
    scratch_types=[
        pltpu.VMEM((_IBLK, _CHUNK), jnp.int32),
        pltpu.VMEM((_CHUNK, 128), jnp.float32),
        pltpu.SemaphoreType.DMA,
        pltpu.VMEM_SHARED((_NA, 128), jnp.float32),
    ],
)
def _deg(dst_hbm, zeros_hbm, ones_hbm, out0, out1, didx, ones_v, sem, acc):
    c = lax.axis_index("c")
    s = lax.axis_index("s")
    _init_acc(zeros_hbm, acc, s)
    pltpu.sync_copy(ones_hbm, ones_v)
    dbase = (c * _NS + s) * _DEG_CHUNKS
    plsc.subcore_barrier()

    @pl.loop(0, _DEG_CHUNKS // _IBLK)
    def _(b):
        pltpu.sync_copy(dst_hbm.at[pl.ds(dbase + b * _IBLK, _IBLK)], didx)

        @pl.loop(0, _IBLK)
        def _(j):
            pltpu.async_copy(ones_v, acc.at[didx.at[j]], sem, add=True)

        @pl.loop(0, _IBLK)
        def _(j):
            pltpu.make_async_copy(ones_v, acc.at[didx.at[j]], sem).wait()

    plsc.subcore_barrier()
    _drain_acc(acc, out0, out1, c, s)


def _scale_body(d0, d1, x, p):
    deg = d0[:, :1] + d1[:, :1] + 1.0
    p[...] = x[...] * lax.rsqrt(deg)


def _mid_body(a0, a1, p, d0, d1, w1, b1, w2, q):
    r = lax.rsqrt(d0[:, :1] + d1[:, :1] + 1.0)
    u = (a0[...] + a1[...] + p[...]) * r
    h = jnp.dot(u, w1[...], preferred_element_type=jnp.float32) + b1[...]
    h = jnp.maximum(h, 0.0)
    qv = jnp.dot(h, w2[...], preferred_element_type=jnp.float32) * r
    q[...] = qv.reshape(2 * _MBLK, 128)


def _final_body(a0, a1, q, d0, d1, b2, o):
    r = lax.rsqrt(d0[:, :1] + d1[:, :1] + 1.0)
    qv = q[...].reshape(_MBLK, _D_OUT_PAD)
    acc = jnp.concatenate([a0[...], a1[...]], axis=1) + qv
    full = jnp.maximum(acc * r + b2[...], 0.0)
    o[...] = full[:, :_D_OUT]


def _row_spec(w):
    return pl.BlockSpec((_MBLK, w), lambda i: (i, 0))


def _full_spec(h, w):
    return pl.BlockSpec((h, w), lambda i: (0, 0))


_GRID = (_N // _MBLK,)

_scale = pl.pallas_call(
    _scale_body,
    grid=_GRID,
    in_specs=[_row_spec(128), _row_spec(128), _row_spec(128)],
    out_specs=_row_spec(128),
    out_shape=jax.ShapeDtypeStruct((_N, 128), jnp.float32),
)

_mid = pl.pallas_call(
    _mid_body,
    grid=_GRID,
    in_specs=[
        _row_spec(128), _row_spec(128), _row_spec(128),
        _row_spec(128), _row_spec(128),
        _full_spec(_D_IN, _D_HID), _full_spec(1, _D_HID),
        _full_spec(_D_HID, _D_OUT_PAD),
    ],
    out_specs=pl.BlockSpec((2 * _MBLK, 128), lambda i: (i, 0)),
    out_shape=jax.ShapeDtypeStruct((2 * _N, 128), jnp.float32),
)

_final = pl.pallas_call(
    _final_body,
    grid=_GRID,
    in_specs=[
        _row_spec(128), _row_spec(128),
        pl.BlockSpec((2 * _MBLK, 128), lambda i: (i, 0)),
        _row_spec(128), _row_spec(128),
        _full_spec(1, _D_OUT_PAD),
    ],
    out_specs=_row_spec(_D_OUT),
    out_shape=jax.ShapeDtypeStruct((_N, _D_OUT), jnp.float32),
)


def kernel(x, edge_index, W1, b1, W2, b2):
    src = edge_index[0].astype(jnp.int32)
    dst = edge_index[1].astype(jnp.int32)
    pad = _EP - _E
    spread = jnp.arange(pad, dtype=jnp.int32)
    src_p = jnp.concatenate([src, spread % _N])
    dst_p = jnp.concatenate([dst, _N + spread % _NTRASH])
    src_r = src_p.reshape(_EP // _CHUNK, _CHUNK)
    dst_r = dst_p.reshape(_EP // _CHUNK, _CHUNK)
    src2_r = jnp.concatenate([2 * src_p, 2 * src_p + 1]).reshape(
        2 * _EP // _CHUNK, _CHUNK)
    zeros128 = jnp.zeros((_NA, 128), jnp.float32)
    ones128 = jnp.ones((_CHUNK, 128), jnp.float32)

    d0, d1 = _deg(dst_r, zeros128, ones128)
    p = _scale(d0, d1, x)
    a0, a1 = _agg1(p, src_r, dst_r, zeros128)
    q_all = _mid(a0, a1, p, d0, d1, W1, b1.reshape(1, _D_HID),
                 jnp.pad(W2, ((0, 0), (0, _D_OUT_PAD - _D_OUT))))
    c0, c1 = _agg2(q_all, src2_r, dst_r, zeros128)
    out = _final(c0, c1, q_all, d0, d1,
                 jnp.pad(b2, (0, _D_OUT_PAD - _D_OUT)).reshape(1, _D_OUT_PAD))
    return out
